# baseline (device time: 17878 ns/iter reference)
import jax
import jax.numpy as jnp
from jax import lax
from jax.experimental import pallas as pl
from jax.experimental.pallas import tpu as pltpu

N_DEV = 4
P = 288
M, N = 1024, 512
DR, DC = 8, 128


def _a2av(x, dr):
    def body(x_ref, d_ref, out_ref,
             x_bf, q_buf, blocks, blk_recv, dest_recv,
             sd_send, sd_recv, sb_send, sb_recv):
        me = lax.axis_index("i")

        barrier = pltpu.get_barrier_semaphore()
        for o in range(1, N_DEV):
            peer = (me + o) % N_DEV
            pl.semaphore_signal(
                barrier, inc=1,
                device_id=(peer,), device_id_type=pl.DeviceIdType.MESH,
            )
        pl.semaphore_wait(barrier, N_DEV - 1)

        dest_rdmas = []
        for o in range(1, N_DEV):
            peer = (me + o) % N_DEV
            rd = pltpu.make_async_remote_copy(
                src_ref=d_ref, dst_ref=dest_recv.at[o - 1],
                send_sem=sd_send.at[o - 1], recv_sem=sd_recv.at[o - 1],
                device_id=(peer,), device_id_type=pl.DeviceIdType.MESH,
            )
            rd.start()
            dest_rdmas.append(rd)

        x_bf[...] = x_ref[...].astype(jnp.bfloat16)

        u128 = (lax.broadcasted_iota(jnp.int32, (DC, DC), 0)
                < lax.broadcasted_iota(jnp.int32, (DC, DC), 1)).astype(
                    jnp.float32)
        s8 = (lax.broadcasted_iota(jnp.int32, (DR, DR), 0)
              > lax.broadcasted_iota(jnp.int32, (DR, DR), 1)).astype(
                  jnp.float32)
        qi_col = lax.broadcasted_iota(
            jnp.int32, (P, DC), 0).astype(jnp.float32)

        dvals = d_ref[...]
        blk_rdmas = {}
        for o in (2, 1, 3, 0):
            d = (me + o) % N_DEV
            mask_d = (dvals == d).astype(jnp.float32)
            within = jax.lax.dot(mask_d, u128,
                                 preferred_element_type=jnp.float32)
            rows_d = jnp.sum(mask_d, axis=1, keepdims=True)
            rowpre = jax.lax.dot(s8, rows_d,
                                 preferred_element_type=jnp.float32)
            rank_d = within + rowpre
            for r in range(DR):
                chunk = ((rank_d[r:r + 1, :] == qi_col)
                         & (mask_d[r:r + 1, :] > 0.0))
                q_buf[o, :, r * DC:(r + 1) * DC] = chunk.astype(jnp.bfloat16)
            blocks[o] = jax.lax.dot(
                q_buf[o], x_bf[...],
                preferred_element_type=jnp.float32).astype(jnp.bfloat16)
            if o:
                rb = pltpu.make_async_remote_copy(
                    src_ref=blocks.at[o], dst_ref=blk_recv.at[o - 1],
                    send_sem=sb_send.at[o - 1], recv_sem=sb_recv.at[o - 1],
                    device_id=(d,), device_id_type=pl.DeviceIdType.MESH,
                )
                rb.start()
                blk_rdmas[o] = rb

        for rd in dest_rdmas:
            rd.wait_recv()

        srcs = [me] + [(me - k - 1) % N_DEV for k in range(N_DEV - 1)]
        cnts = [jnp.sum((dvals == me).astype(jnp.int32))] + [
            jnp.sum((dest_recv[k] == me).astype(jnp.int32))
            for k in range(N_DEV - 1)
        ]
        bases = []
        for i in range(N_DEV):
            b = jnp.int32(0)
            for j in range(N_DEV):
                if j != i:
                    b = b + jnp.where(srcs[j] < srcs[i], cnts[j], 0)
            bases.append(b)

        pi = lax.broadcasted_iota(jnp.int32, (M, P), 0)
        qi = lax.broadcasted_iota(jnp.int32, (M, P), 1)

        def shift_mat(i):
            return ((pi - qi == bases[i]) & (qi < cnts[i])).astype(
                jnp.bfloat16)

        acc = jax.lax.dot(shift_mat(0), blocks[0],
                          preferred_element_type=jnp.float32)
        for o in (1, 3, 2):
            blk_rdmas[o].wait_recv()
            acc = acc + jax.lax.dot(shift_mat(o), blk_recv[o - 1],
                                    preferred_element_type=jnp.float32)
        out_ref[...] = acc.astype(jnp.bfloat16)

        for rd in dest_rdmas:
            rd.wait_send()
        for rb in blk_rdmas.values():
            rb.wait_send()

    return pl.pallas_call(
        body,
        out_shape=jax.ShapeDtypeStruct((M, N), jnp.bfloat16),
        in_specs=[
            pl.BlockSpec(memory_space=pltpu.VMEM),
            pl.BlockSpec(memory_space=pltpu.VMEM),
        ],
        out_specs=pl.BlockSpec(memory_space=pltpu.VMEM),
        scratch_shapes=[
            pltpu.VMEM((M, N), jnp.bfloat16),
            pltpu.VMEM((N_DEV, P, M), jnp.bfloat16),
            pltpu.VMEM((N_DEV, P, N), jnp.bfloat16),
            pltpu.VMEM((N_DEV - 1, P, N), jnp.bfloat16),
            pltpu.VMEM((N_DEV - 1, DR, DC), jnp.int32),
            pltpu.SemaphoreType.DMA((N_DEV - 1,)),
            pltpu.SemaphoreType.DMA((N_DEV - 1,)),
            pltpu.SemaphoreType.DMA((N_DEV - 1,)),
            pltpu.SemaphoreType.DMA((N_DEV - 1,)),
        ],
        compiler_params=pltpu.CompilerParams(collective_id=0),
    )(x, dr)


def kernel(x, dest):
    return _a2av(x, dest.reshape(DR, DC))
